# Initial kernel scaffold; baseline (speedup 1.0000x reference)
import jax
import jax.numpy as jnp
from jax.experimental import pallas as pl


def kernel(fp_data, mol_x, mol_edge_index, mol_batch, kg_edge_index, kg_edge_type, fp_W1, fp_b1, fp_g1, fp_be1, fp_W2, fp_b2, fp_g2, fp_be2, Wg1, bg1, g_g1, g_be1, Wg2, bg2, g_g2, g_be2, att_W1, att_b1, att_W2, gene_emb, Wrel1, Wroot1, brg1, kg_g1, kg_be1, Wrel2, Wroot2, brg2, kg_g2, kg_be2, W_l1, b_l1, kg_g3, kg_be3, W_l2, b_l2):
    raise NotImplementedError("write your pallas kernel here")


# speedup vs baseline: 2.2751x; 2.2751x over previous
import functools

import jax
import jax.numpy as jnp
from jax import lax
from jax.experimental import pallas as pl
from jax.experimental.pallas import tpu as pltpu
from jax.experimental.pallas import tpu_sc as plsc

N_DRUG = 4586
N_GENE = 5414
N_KG = N_DRUG + N_GENE
N_MOL = 100000
E_MOL = 400000
E_KG = 320000
N_REL = 8
EPS = 1e-5

DP = 4608
MP = 100352
KP = 10240
RKG = N_REL * KP

EP_MOL = 425984
EP_KG = 327680
EP_POOL = 131072

SC_B = 4096
SC_A = SC_B + 256
SC_TRASH = SC_B + 8

_mesh = plsc.VectorSubcoreMesh(core_axis_name="c", subcore_axis_name="s")
_sc_params = pltpu.CompilerParams(needs_layout_passes=False)


def _sc_hist(idx2d, r_out):
    rh = (r_out // 2048 + 1) * 2048
    r128 = rh // 128
    nbat = r128 // 16
    nrows = idx2d.shape[0]
    nbpw = nrows // 32

    @functools.partial(
        pl.kernel,
        out_type=jax.ShapeDtypeStruct((2, r128, 128), jnp.float32),
        mesh=_mesh,
        compiler_params=_sc_params,
        scratch_types=[
            pltpu.VMEM((nbpw, 128), jnp.int32),
            pltpu.VMEM((r128, 128), jnp.float32),
            pltpu.VMEM((16, 128), jnp.float32),
            pltpu.VMEM((1, 16), jnp.int32),
            pltpu.VMEM_SHARED((r128, 128), jnp.float32),
            pltpu.SemaphoreType.DMA,
        ],
    )
    def hist_kernel(i_hbm, out_hbm, ebuf, lhist, zbuf, stage, hsh, sem):
        cid = lax.axis_index("c")
        sid = lax.axis_index("s")
        wid = cid * 16 + sid

        @pl.loop(0, 16)
        def _(i):
            for k in range(8):
                zbuf[i, pl.ds(k * 16, 16)] = jnp.zeros((16,), jnp.float32)

        @pl.loop(0, r128)
        def _(i):
            for k in range(8):
                lhist[i, pl.ds(k * 16, 16)] = jnp.zeros((16,), jnp.float32)

        pltpu.sync_copy(i_hbm.at[pl.ds(wid * nbpw, nbpw)], ebuf)

        @pl.when(sid == 0)
        def _():
            @pl.loop(0, r128 // 16)
            def _(i):
                pltpu.sync_copy(zbuf, hsh.at[pl.ds(i * 16, 16)])

        ones = jnp.ones((16,), jnp.float32)

        @pl.loop(0, nbpw)
        def _(j):
            for k in range(8):
                v = ebuf[j, pl.ds(k * 16, 16)]
                plsc.addupdate_scatter(
                    lhist, [lax.shift_right_logical(v, 7), v & 127], ones)

        plsc.subcore_barrier()

        skew = nbat // 16 + 1

        @pl.loop(0, nbat)
        def _(b):
            bb = lax.rem(b + sid * skew, nbat)
            stage[0, pl.ds(0, 16)] = (
                jnp.arange(16, dtype=jnp.int32) + bb * 16)
            pltpu.sync_copy(lhist.at[pl.ds(bb * 16, 16)],
                            hsh.at[stage.at[0]], add=True)

        plsc.subcore_barrier()

        @pl.when(sid == 0)
        def _():
            pltpu.sync_copy(hsh, out_hbm.at[cid])

    return hist_kernel(idx2d)


def _sc_scatter(table, src2d, dst2d, r_out, npass):
    nrows = src2d.shape[0]
    nbpw = nrows // 32
    epw = nbpw * 128
    cap = epw + 32

    @functools.partial(
        pl.kernel,
        out_type=jax.ShapeDtypeStruct((2, r_out, 128), jnp.float32),
        mesh=_mesh,
        compiler_params=_sc_params,
        scratch_types=[
            pltpu.VMEM((nbpw, 128), jnp.int32),
            pltpu.VMEM((nbpw, 128), jnp.int32),
            pltpu.VMEM((cap,), jnp.int32),
            pltpu.VMEM((cap,), jnp.int32),
            pltpu.VMEM((128, 128), jnp.float32),
            pltpu.VMEM((128, 128), jnp.float32),
            pltpu.VMEM((1, 128), jnp.int32),
            pltpu.VMEM_SHARED((SC_A, 128), jnp.float32),
            pltpu.SemaphoreType.DMA,
        ],
    )
    def scat_kernel(tab_hbm, src_hbm, dst_hbm, out_hbm,
                    esrc, edst, csrc, cdst, rows, zbuf, stage, acc, sem):
        cid = lax.axis_index("c")
        sid = lax.axis_index("s")
        wid = cid * 16 + sid

        @pl.loop(0, 128)
        def _(i):
            for k in range(8):
                zbuf[i, pl.ds(k * 16, 16)] = jnp.zeros((16,), jnp.float32)

        pltpu.sync_copy(src_hbm.at[pl.ds(wid * nbpw, nbpw)], esrc)
        pltpu.sync_copy(dst_hbm.at[pl.ds(wid * nbpw, nbpw)], edst)

        @pl.loop(0, cap // 16)
        def _(i):
            csrc[pl.ds(i * 16, 16)] = jnp.zeros((16,), jnp.int32)

        @pl.loop(0, npass)
        def _pass(p):
            base = p * SC_B
            valid = jnp.minimum(r_out - base, SC_B)

            rpt = SC_A // 16
            s0 = sid * rpt
            for c in range(rpt // 128):
                pltpu.sync_copy(zbuf, acc.at[pl.ds(s0 + c * 128, 128)])
            if rpt % 128:
                pltpu.sync_copy(zbuf.at[pl.ds(0, rpt % 128)],
                                acc.at[pl.ds(s0 + rpt - rpt % 128, rpt % 128)])

            @pl.loop(0, cap // 16)
            def _(i):
                cdst[pl.ds(i * 16, 16)] = jnp.full((16,), SC_TRASH, jnp.int32)

            plsc.subcore_barrier()

            def cstep(j, n):
                for k in range(8):
                    d = edst[j, pl.ds(k * 16, 16)]
                    s = esrc[j, pl.ds(k * 16, 16)]
                    m = (d >= base) & (d < base + SC_B)
                    plsc.store_compressed(cdst.at[pl.ds(n, 16)], d - base, mask=m)
                    plsc.store_compressed(csrc.at[pl.ds(n, 16)], s, mask=m)
                    n = n + jnp.sum(m.astype(jnp.int32))
                return n

            n = lax.fori_loop(0, nbpw, cstep, jnp.int32(0))
            nb = (n + 127) // 128

            def bstep(j, carry):
                for k in range(8):
                    stage[0, pl.ds(k * 16, 16)] = cdst[pl.ds(j * 128 + k * 16, 16)]
                pltpu.sync_copy(tab_hbm.at[csrc.at[pl.ds(j * 128, 128)]], rows)
                pltpu.sync_copy(rows, acc.at[stage.at[0]], add=True)
                return carry

            lax.fori_loop(0, nb, bstep, jnp.int32(0))
            plsc.subcore_barrier()

            o0 = sid * (SC_B // 16)

            @pl.when(o0 < valid)
            def _():
                pltpu.sync_copy(
                    acc.at[pl.ds(o0, SC_B // 16)],
                    out_hbm.at[cid, pl.ds(base + o0, SC_B // 16)],
                )

            plsc.subcore_barrier()

    return scat_kernel(table, src2d, dst2d)


def _mm(x, w, bias=None, row_scale=None, bm=512):
    m, kdim = x.shape
    n = w.shape[1]
    in_specs = [
        pl.BlockSpec((bm, kdim), lambda i: (i, 0)),
        pl.BlockSpec((kdim, n), lambda i: (0, 0)),
    ]
    args = [x, w]
    if bias is not None:
        in_specs.append(pl.BlockSpec((1, n), lambda i: (0, 0)))
        args.append(bias.reshape(1, n))
    if row_scale is not None:
        in_specs.append(pl.BlockSpec((bm, 1), lambda i: (i, 0)))
        args.append(row_scale)

    def body(*refs):
        x_ref, w_ref = refs[0], refs[1]
        o_ref = refs[-1]
        y = jnp.dot(x_ref[...], w_ref[...], preferred_element_type=jnp.float32)
        idx = 2
        if bias is not None:
            y = y + refs[idx][...]
            idx += 1
        if row_scale is not None:
            y = y * refs[idx][...]
        o_ref[...] = y

    return pl.pallas_call(
        body,
        grid=(m // bm,),
        in_specs=in_specs,
        out_specs=pl.BlockSpec((bm, n), lambda i: (i, 0)),
        out_shape=jax.ShapeDtypeStruct((m, n), jnp.float32),
    )(*args)


def _mm_rel(x, wrel, bm=512):
    m = x.shape[0]
    n = wrel.shape[2]

    def body(x_ref, w_ref, o_ref):
        o_ref[0] = jnp.dot(x_ref[...], w_ref[0],
                           preferred_element_type=jnp.float32)

    return pl.pallas_call(
        body,
        grid=(N_REL, m // bm),
        in_specs=[
            pl.BlockSpec((bm, x.shape[1]), lambda r, i: (i, 0)),
            pl.BlockSpec((1, wrel.shape[1], n), lambda r, i: (r, 0, 0)),
        ],
        out_specs=pl.BlockSpec((1, bm, n), lambda r, i: (r, i, 0)),
        out_shape=jax.ShapeDtypeStruct((N_REL, m, n), jnp.float32),
    )(x, wrel)


def _bn_relu_generic(args, specs, combine, m, m_real, n, g, be, bm=512):
    nblk = m // bm
    all_args = list(args) + [g.reshape(1, n), be.reshape(1, n)]
    all_specs = list(specs) + [
        pl.BlockSpec((1, n), lambda p, i: (0, 0)),
        pl.BlockSpec((1, n), lambda p, i: (0, 0)),
    ]

    def body(*refs):
        *in_refs, g_ref, be_ref, o_ref, s_ref, q_ref = refs
        p = pl.program_id(0)
        i = pl.program_id(1)
        y = combine(*[r[...] for r in in_refs])
        rowid = lax.broadcasted_iota(jnp.int32, (bm, 1), 0) + i * bm
        msk = (rowid < m_real).astype(jnp.float32)
        ym = y * msk

        @pl.when((p == 0) & (i == 0))
        def _():
            s_ref[...] = jnp.zeros_like(s_ref)
            q_ref[...] = jnp.zeros_like(q_ref)

        @pl.when(p == 0)
        def _():
            s_ref[...] += jnp.sum(ym, axis=0, keepdims=True)
            q_ref[...] += jnp.sum(ym * ym, axis=0, keepdims=True)
            o_ref[...] = ym

        @pl.when(p == 1)
        def _():
            mean = s_ref[...] / m_real
            var = q_ref[...] / m_real - mean * mean
            inv = g_ref[...] * lax.rsqrt(var + EPS)
            o_ref[...] = jnp.maximum((y - mean) * inv + be_ref[...], 0.0) * msk

    return pl.pallas_call(
        body,
        grid=(2, nblk),
        in_specs=all_specs,
        out_specs=pl.BlockSpec((bm, n), lambda p, i: (i, 0)),
        out_shape=jax.ShapeDtypeStruct((m, n), jnp.float32),
        scratch_shapes=[
            pltpu.VMEM((1, n), jnp.float32),
            pltpu.VMEM((1, n), jnp.float32),
        ],
    )(*all_args)


def _bn_relu(y, m_real, g, be):
    m, n = y.shape
    return _bn_relu_generic(
        [y], [pl.BlockSpec((512, n), lambda p, i: (i, 0))],
        lambda yy: yy, m, m_real, n, g, be)


def _bn_mol(agg, hp, dinv, b, g, be):
    m = hp.shape[0]
    specs = [
        pl.BlockSpec((2, 512, 128), lambda p, i: (0, i, 0)),
        pl.BlockSpec((512, 128), lambda p, i: (i, 0)),
        pl.BlockSpec((512, 1), lambda p, i: (i, 0)),
        pl.BlockSpec((1, 128), lambda p, i: (0, 0)),
    ]

    def combine(a, hpv, dv, bv):
        return (a[0] + a[1] + hpv) * dv + bv

    return _bn_relu_generic([agg, hp, dinv, b.reshape(1, 128)], specs,
                            combine, m, N_MOL, 128, g, be)


def _bn_kg(aggk, invc, root, b, g, be):
    m = root.shape[0]
    specs = [
        pl.BlockSpec((2, N_REL, 512, 128), lambda p, i: (0, 0, i, 0)),
        pl.BlockSpec((N_REL, 512, 1), lambda p, i: (0, i, 0)),
        pl.BlockSpec((512, 128), lambda p, i: (i, 0)),
        pl.BlockSpec((1, 128), lambda p, i: (0, 0)),
    ]

    def combine(a, ic, rt, bv):
        y = rt + bv
        for r in range(N_REL):
            y = y + (a[0, r] + a[1, r]) * ic[r]
        return y

    return _bn_relu_generic([aggk, invc, root, b.reshape(1, 128)], specs,
                            combine, m, N_KG, 128, g, be)


def _colvec(hist2, kind):
    two, r, _ = hist2.shape

    def body(h_ref, o_ref):
        v = h_ref[0] + h_ref[1]
        if kind == "rsqrt1":
            v = lax.rsqrt(v + 1.0)
        else:
            v = 1.0 / jnp.maximum(v, 1.0)
        o_ref[...] = v

    out = pl.pallas_call(
        body,
        out_shape=jax.ShapeDtypeStruct((r, 128), jnp.float32),
    )(hist2)
    return out.reshape(r * 128, 1)


def _flat_idx(src2d, dst2d, et2d):
    def body(s_ref, d_ref, t_ref, os_ref, od_ref):
        t = t_ref[...]
        os_ref[...] = jnp.minimum(t, N_REL - 1) * KP + s_ref[...]
        od_ref[...] = t * KP + d_ref[...]

    return pl.pallas_call(
        body,
        out_shape=(
            jax.ShapeDtypeStruct(src2d.shape, jnp.int32),
            jax.ShapeDtypeStruct(src2d.shape, jnp.int32),
        ),
    )(src2d, dst2d, et2d)


def _attention(sums, invc_d, fpx, w1, b1, w2):
    def body(s_ref, ic_ref, f_ref, w1_ref, b1_ref, w2_ref, emb_ref, beta_ref):
        gmean = (s_ref[0] + s_ref[1]) * ic_ref[...]
        f = f_ref[...]
        w1v = w1_ref[...]
        b1v = b1_ref[...]
        w2v = w2_ref[...]
        s1 = jnp.dot(jnp.tanh(jnp.dot(gmean, w1v,
                                      preferred_element_type=jnp.float32) + b1v),
                     w2v, preferred_element_type=jnp.float32)
        s2 = jnp.dot(jnp.tanh(jnp.dot(f, w1v,
                                      preferred_element_type=jnp.float32) + b1v),
                     w2v, preferred_element_type=jnp.float32)
        mx = jnp.maximum(s1, s2)
        e1 = jnp.exp(s1 - mx)
        e2 = jnp.exp(s2 - mx)
        den = e1 + e2
        be1 = e1 / den
        be2 = e2 / den
        emb_ref[...] = gmean * be1 + f * be2
        beta_ref[...] = jnp.concatenate([be1, be2], axis=1)

    return pl.pallas_call(
        body,
        out_shape=(
            jax.ShapeDtypeStruct((DP, 128), jnp.float32),
            jax.ShapeDtypeStruct((DP, 2), jnp.float32),
        ),
    )(sums, invc_d, fpx, w1, b1.reshape(1, 128), w2)


def _final(x, w, b):
    def body(x_ref, w_ref, b_ref, o_ref):
        y = jnp.dot(x_ref[...], w_ref[...],
                    preferred_element_type=jnp.float32) + b_ref[...]
        mx = jnp.max(y, axis=1, keepdims=True)
        e = jnp.exp(y - mx)
        lse = mx + jnp.log(jnp.sum(e, axis=1, keepdims=True))
        o_ref[...] = y - lse

    return pl.pallas_call(
        body,
        out_shape=jax.ShapeDtypeStruct((x.shape[0], 2), jnp.float32),
    )(x, w, b.reshape(1, 2))


def kernel(fp_data, mol_x, mol_edge_index, mol_batch, kg_edge_index,
           kg_edge_type, fp_W1, fp_b1, fp_g1, fp_be1, fp_W2, fp_b2, fp_g2,
           fp_be2, Wg1, bg1, g_g1, g_be1, Wg2, bg2, g_g2, g_be2, att_W1,
           att_b1, att_W2, gene_emb, Wrel1, Wroot1, brg1, kg_g1, kg_be1,
           Wrel2, Wroot2, brg2, kg_g2, kg_be2, W_l1, b_l1, kg_g3, kg_be3,
           W_l2, b_l2):
    fp_pad = jnp.pad(fp_data, ((0, DP - N_DRUG), (0, 0)))
    y1 = _mm(fp_pad, fp_W1, bias=fp_b1)
    fx1 = _bn_relu(y1, N_DRUG, fp_g1, fp_be1)
    y2 = _mm(fx1, fp_W2, bias=fp_b2)
    fpx = _bn_relu(y2, N_DRUG, fp_g2, fp_be2)

    msrc = jnp.pad(mol_edge_index[0], (0, EP_MOL - E_MOL)).reshape(-1, 128)
    mdst = jnp.pad(mol_edge_index[1], (0, EP_MOL - E_MOL),
                   constant_values=MP).reshape(-1, 128)
    hist_m = _sc_hist(mdst, MP).reshape(2, -1)
    dinv = _colvec(hist_m[:, :MP].reshape(2, -1, 128), "rsqrt1")

    x0 = jnp.pad(mol_x, ((0, MP - N_MOL), (0, 0)))
    hp1 = _mm(x0, Wg1, row_scale=dinv)
    agg1 = _sc_scatter(hp1, msrc, mdst, MP, 25)
    x1 = _bn_mol(agg1, hp1, dinv, bg1, g_g1, g_be1)
    hp2 = _mm(x1, Wg2, row_scale=dinv)
    agg2 = _sc_scatter(hp2, msrc, mdst, MP, 25)
    x2 = _bn_mol(agg2, hp2, dinv, bg2, g_g2, g_be2)

    ar = jnp.arange(EP_POOL, dtype=jnp.int32)
    psrc = jnp.where(ar < N_MOL, ar, 0).reshape(-1, 128)
    pdst = jnp.pad(mol_batch, (0, EP_POOL - N_MOL),
                   constant_values=DP).reshape(-1, 128)
    sums = _sc_scatter(x2, psrc, pdst, DP, 2)
    hist_p = _sc_hist(pdst, DP).reshape(2, -1)
    invc_d = _colvec(hist_p[:, :DP].reshape(2, -1, 128), "invmax1")

    emb, beta = _attention(sums, invc_d, fpx, att_W1, att_b1, att_W2)

    kx = jnp.pad(jnp.concatenate([emb[:N_DRUG], gene_emb], axis=0),
                 ((0, KP - N_KG), (0, 0)))
    ksrc = jnp.pad(kg_edge_index[0], (0, EP_KG - E_KG)).reshape(-1, 128)
    kdst = jnp.pad(kg_edge_index[1], (0, EP_KG - E_KG)).reshape(-1, 128)
    ket = jnp.pad(kg_edge_type, (0, EP_KG - E_KG),
                  constant_values=N_REL).reshape(-1, 128)
    sflat, dflat = _flat_idx(ksrc, kdst, ket)
    hist_k = _sc_hist(dflat, RKG).reshape(2, -1)
    invc_k = _colvec(hist_k[:, :RKG].reshape(2, -1, 128),
                     "invmax1").reshape(N_REL, KP, 1)

    def rgcn_layer(kxi, wrel, wroot, brg, gg, bb):
        hr = _mm_rel(kxi, wrel)
        aggk = _sc_scatter(hr.reshape(RKG, 128), sflat, dflat, RKG, 20)
        root = _mm(kxi, wroot)
        return _bn_kg(aggk.reshape(2, N_REL, KP, 128), invc_k, root,
                      brg, gg, bb)

    k1 = rgcn_layer(kx, Wrel1, Wroot1, brg1, kg_g1, kg_be1)
    k2 = rgcn_layer(k1, Wrel2, Wroot2, brg2, kg_g2, kg_be2)

    y3 = _mm(k2, W_l1, bias=b_l1)
    k3 = _bn_relu(y3, N_KG, kg_g3, kg_be3)
    outp = _final(k3, W_l2, b_l2)

    return (outp[:N_KG], beta[:N_DRUG][:, :, None])
